# baseline (speedup 1.0000x reference)
import dataclasses

import jax
import jax.numpy as jnp
from jax import lax
from jax.experimental import pallas as pl
from jax.experimental.pallas import tpu as pltpu
from jax.experimental.pallas import tpu_sc as plsc

VOCAB = 30
VOCAB_P = 32
OUT_LEN = 256
SEQ = 50
BLOCK_B = 512
N_TEC = 32
ROWS_PER_TEC = 4096 // N_TEC

_VECTOR_MESH = plsc.VectorSubcoreMesh(core_axis_name="c", subcore_axis_name="s")

_SC_CP = pltpu.CompilerParams()
if "needs_layout_passes" in pltpu.CompilerParams.__dataclass_fields__:
    _SC_CP = dataclasses.replace(_SC_CP, needs_layout_passes=False)


def _sc_hist(tok_ref, counts_ref, tok_vmem, cnt_vmem, sem):
    c = lax.axis_index("c")
    s = lax.axis_index("s")
    tec = c * 16 + s
    r0 = tec * ROWS_PER_TEC
    cp_in = pltpu.make_async_copy(
        tok_ref.at[pl.ds(r0, ROWS_PER_TEC), :], tok_vmem, sem)
    cp_in.start()
    cp_in.wait()
    zeros = jnp.zeros((16,), jnp.float32)

    @pl.loop(0, ROWS_PER_TEC)
    def _(r):
        cnt_vmem[r, pl.ds(0, 16)] = zeros
        cnt_vmem[r, pl.ds(16, 16)] = zeros

    ones = jnp.full((16,), 1.0, jnp.float32)
    iota16 = lax.broadcasted_iota(jnp.int32, (16,), 0)

    @pl.loop(0, ROWS_PER_TEC // 16)
    def _(g):
        ridx = g * 16 + iota16

        @pl.loop(0, SEQ)
        def _(l):
            lidx = jnp.full((16,), l, jnp.int32)
            t16 = plsc.load_gather(tok_vmem, [ridx, lidx])
            plsc.addupdate_scatter(cnt_vmem, [ridx, t16], ones)

    cp_out = pltpu.make_async_copy(
        cnt_vmem, counts_ref.at[pl.ds(r0, ROWS_PER_TEC), :], sem)
    cp_out.start()
    cp_out.wait()


def _tc_body(cnt_ref, tt_ref, out_ref):
    counts = cnt_ref[...].astype(jnp.bfloat16)
    for v in range(VOCAB):
        out_ref[v, :, :] = jnp.dot(counts, tt_ref[v],
                                   preferred_element_type=jnp.float32)


@jax.jit
def kernel(tokens, table):
    batch = tokens.shape[0]
    tokens = tokens.astype(jnp.int32)

    counts = pl.kernel(
        _sc_hist,
        out_type=jax.ShapeDtypeStruct((batch, VOCAB_P), jnp.float32),
        mesh=_VECTOR_MESH,
        scratch_types=[
            pltpu.VMEM((ROWS_PER_TEC, SEQ), jnp.int32),
            pltpu.VMEM((ROWS_PER_TEC, VOCAB_P), jnp.float32),
            pltpu.SemaphoreType.DMA,
        ],
        compiler_params=_SC_CP,
    )(tokens)

    tt = table.reshape(VOCAB, OUT_LEN, VOCAB).transpose(2, 0, 1)
    tt = jnp.pad(tt, ((0, 0), (0, VOCAB_P - VOCAB), (0, 0)))
    tt = tt.astype(jnp.bfloat16)

    grid = (batch // BLOCK_B,)
    out_t = pl.pallas_call(
        _tc_body,
        grid=grid,
        in_specs=[
            pl.BlockSpec((BLOCK_B, VOCAB_P), lambda i: (i, 0)),
            pl.BlockSpec((VOCAB, VOCAB_P, OUT_LEN), lambda i: (0, 0, 0)),
        ],
        out_specs=pl.BlockSpec((VOCAB, BLOCK_B, OUT_LEN), lambda i: (0, i, 0)),
        out_shape=jax.ShapeDtypeStruct((VOCAB, batch, OUT_LEN), jnp.float32),
        compiler_params=pltpu.CompilerParams(
            dimension_semantics=("parallel",),
        ),
    )(counts, tt)
    return out_t.transpose(1, 2, 0)


# speedup vs baseline: 1.0634x; 1.0634x over previous
import dataclasses

import jax
import jax.numpy as jnp
from jax import lax
from jax.experimental import pallas as pl
from jax.experimental.pallas import tpu as pltpu
from jax.experimental.pallas import tpu_sc as plsc

VOCAB = 30
VOCAB_P = 32
OUT_LEN = 256
SEQ = 50
BLOCK_B = 512
N_TEC = 32
ROWS_PER_TEC = 4096 // N_TEC

_VECTOR_MESH = plsc.VectorSubcoreMesh(core_axis_name="c", subcore_axis_name="s")

_SC_CP = pltpu.CompilerParams()
if "needs_layout_passes" in pltpu.CompilerParams.__dataclass_fields__:
    _SC_CP = dataclasses.replace(_SC_CP, needs_layout_passes=False)


def _sc_hist(tokt_ref, counts_ref, tok_vmem, cnt_vmem, sem):
    c = lax.axis_index("c")
    s = lax.axis_index("s")
    tec = c * 16 + s
    r0 = tec * ROWS_PER_TEC
    cp_in = pltpu.make_async_copy(
        tokt_ref.at[:, pl.ds(r0, ROWS_PER_TEC)], tok_vmem, sem)
    cp_in.start()
    cp_in.wait()
    zeros = jnp.zeros((16,), jnp.float32)

    @pl.loop(0, ROWS_PER_TEC)
    def _(r):
        cnt_vmem[r, pl.ds(0, 16)] = zeros
        cnt_vmem[r, pl.ds(16, 16)] = zeros

    ones = jnp.full((16,), 1.0, jnp.float32)
    iota16 = lax.broadcasted_iota(jnp.int32, (16,), 0)

    @pl.loop(0, ROWS_PER_TEC // 16)
    def _(g):
        ridx = g * 16 + iota16

        @pl.loop(0, SEQ)
        def _(l):
            t16 = tok_vmem[l, pl.ds(g * 16, 16)]
            plsc.addupdate_scatter(cnt_vmem, [ridx, t16], ones)

    cp_out = pltpu.make_async_copy(
        cnt_vmem, counts_ref.at[pl.ds(r0, ROWS_PER_TEC), :], sem)
    cp_out.start()
    cp_out.wait()


def _tc_body(cnt_ref, tt_ref, out_ref):
    counts = cnt_ref[...].astype(jnp.bfloat16)
    for v in range(VOCAB):
        out_ref[v, :, :] = jnp.dot(counts, tt_ref[v],
                                   preferred_element_type=jnp.float32)


@jax.jit
def kernel(tokens, table):
    batch = tokens.shape[0]
    tokens = tokens.astype(jnp.int32)
    tokt = tokens.T

    counts = pl.kernel(
        _sc_hist,
        out_type=jax.ShapeDtypeStruct((batch, VOCAB_P), jnp.float32),
        mesh=_VECTOR_MESH,
        scratch_types=[
            pltpu.VMEM((SEQ, ROWS_PER_TEC), jnp.int32),
            pltpu.VMEM((ROWS_PER_TEC, VOCAB_P), jnp.float32),
            pltpu.SemaphoreType.DMA,
        ],
        compiler_params=_SC_CP,
    )(tokt)

    tt = table.reshape(VOCAB, OUT_LEN, VOCAB).transpose(2, 0, 1)
    tt = jnp.pad(tt, ((0, 0), (0, VOCAB_P - VOCAB), (0, 0)))
    tt = tt.astype(jnp.bfloat16)

    grid = (batch // BLOCK_B,)
    out_t = pl.pallas_call(
        _tc_body,
        grid=grid,
        in_specs=[
            pl.BlockSpec((BLOCK_B, VOCAB_P), lambda i: (i, 0)),
            pl.BlockSpec((VOCAB, VOCAB_P, OUT_LEN), lambda i: (0, 0, 0)),
        ],
        out_specs=pl.BlockSpec((VOCAB, BLOCK_B, OUT_LEN), lambda i: (0, i, 0)),
        out_shape=jax.ShapeDtypeStruct((VOCAB, batch, OUT_LEN), jnp.float32),
        compiler_params=pltpu.CompilerParams(
            dimension_semantics=("parallel",),
        ),
    )(counts, tt)
    return out_t.transpose(1, 2, 0)


# speedup vs baseline: 1.0723x; 1.0083x over previous
import dataclasses

import jax
import jax.numpy as jnp
from jax import lax
from jax.experimental import pallas as pl
from jax.experimental.pallas import tpu as pltpu
from jax.experimental.pallas import tpu_sc as plsc

VOCAB = 30
VOCAB_P = 32
OUT_LEN = 256
SEQ = 50
BLOCK_B = 512
N_TEC = 32
ROWS_PER_TEC = 4096 // N_TEC

_VECTOR_MESH = plsc.VectorSubcoreMesh(core_axis_name="c", subcore_axis_name="s")

_SC_CP = pltpu.CompilerParams()
if "needs_layout_passes" in pltpu.CompilerParams.__dataclass_fields__:
    _SC_CP = dataclasses.replace(_SC_CP, needs_layout_passes=False)


def _sc_hist(tokt_ref, counts_ref, tok_vmem, cnt_vmem, sem):
    c = lax.axis_index("c")
    s = lax.axis_index("s")
    tec = c * 16 + s
    r0 = tec * ROWS_PER_TEC
    cp_in = pltpu.make_async_copy(
        tokt_ref.at[:, pl.ds(r0, ROWS_PER_TEC)], tok_vmem, sem)
    cp_in.start()
    zeros = jnp.zeros((16,), jnp.float32)

    @pl.loop(0, ROWS_PER_TEC)
    def _(r):
        cnt_vmem[r, pl.ds(0, 16)] = zeros
        cnt_vmem[r, pl.ds(16, 16)] = zeros

    cp_in.wait()
    ones = jnp.full((16,), 1.0, jnp.float32)
    iota16 = lax.broadcasted_iota(jnp.int32, (16,), 0)

    @pl.loop(0, ROWS_PER_TEC // 16)
    def _(g):
        ridx = g * 16 + iota16

        @pl.loop(0, SEQ)
        def _(l):
            t16 = tok_vmem[l, pl.ds(g * 16, 16)]
            plsc.addupdate_scatter(cnt_vmem, [ridx, t16], ones)

    cp_out = pltpu.make_async_copy(
        cnt_vmem, counts_ref.at[pl.ds(r0, ROWS_PER_TEC), :], sem)
    cp_out.start()
    cp_out.wait()


def _tc_body(cnt_ref, tt_ref, out_ref):
    counts = cnt_ref[...].astype(jnp.bfloat16)
    for v in range(VOCAB):
        out_ref[v, :, :] = jnp.dot(counts, tt_ref[v],
                                   preferred_element_type=jnp.float32)


@jax.jit
def kernel(tokens, table):
    batch = tokens.shape[0]
    tokens = tokens.astype(jnp.int32)
    tokt = tokens.T

    counts = pl.kernel(
        _sc_hist,
        out_type=jax.ShapeDtypeStruct((batch, VOCAB_P), jnp.float32),
        mesh=_VECTOR_MESH,
        scratch_types=[
            pltpu.VMEM((SEQ, ROWS_PER_TEC), jnp.int32),
            pltpu.VMEM((ROWS_PER_TEC, VOCAB_P), jnp.float32),
            pltpu.SemaphoreType.DMA,
        ],
        compiler_params=_SC_CP,
    )(tokt)

    tt = table.reshape(VOCAB, OUT_LEN, VOCAB).transpose(2, 0, 1)
    tt = jnp.pad(tt, ((0, 0), (0, VOCAB_P - VOCAB), (0, 0)))
    tt = tt.astype(jnp.bfloat16)

    grid = (batch // BLOCK_B,)
    out_t = pl.pallas_call(
        _tc_body,
        grid=grid,
        in_specs=[
            pl.BlockSpec((BLOCK_B, VOCAB_P), lambda i: (i, 0)),
            pl.BlockSpec((VOCAB, VOCAB_P, OUT_LEN), lambda i: (0, 0, 0)),
        ],
        out_specs=pl.BlockSpec((VOCAB, BLOCK_B, OUT_LEN), lambda i: (0, i, 0)),
        out_shape=jax.ShapeDtypeStruct((VOCAB, batch, OUT_LEN), jnp.float32),
        compiler_params=pltpu.CompilerParams(
            dimension_semantics=("parallel",),
        ),
    )(counts, tt)
    return out_t.transpose(1, 2, 0)
